# baseline (speedup 1.0000x reference)
import functools

import jax
import jax.numpy as jnp
from jax import lax
from jax.experimental import pallas as pl
from jax.experimental.pallas import tpu as pltpu
from jax.experimental.pallas import tpu_sc as plsc

D = 64
NC, NS = 2, 16
NW = NC * NS
CHUNK = 128
NBUF = 4


@functools.partial(jax.jit, static_argnums=())
def _sc_gather(idx2d, table):
    g_tot, chunk = idx2d.shape
    assert chunk == CHUNK and g_tot % NW == 0
    G = g_tot // NW
    assert G % NBUF == 0
    n_rows = g_tot * CHUNK
    mesh = plsc.VectorSubcoreMesh(core_axis_name="c", subcore_axis_name="s")

    @functools.partial(
        pl.kernel,
        out_type=jax.ShapeDtypeStruct((n_rows, D), jnp.float32),
        mesh=mesh,
        compiler_params=pltpu.CompilerParams(use_tc_tiling_on_sc=False),
        scratch_types=[
            pltpu.VMEM((G, CHUNK), jnp.int32),
            [pltpu.VMEM((CHUNK, D), jnp.float32) for _ in range(NBUF)],
            [pltpu.SemaphoreType.DMA for _ in range(NBUF)],
        ],
    )
    def k(table_hbm, idx_hbm, out_hbm, idx_v, bufs, gsems):
        wid = lax.axis_index("s") * NC + lax.axis_index("c")
        gbase = wid * G
        rbase = gbase * CHUNK
        pltpu.sync_copy(idx_hbm.at[pl.ds(gbase, G)], idx_v)

        def fire(g, b):
            pltpu.make_async_copy(
                table_hbm.at[idx_v.at[g]], bufs[b], gsems[b]
            ).start()

        def drain(g, b):
            pltpu.make_async_copy(
                table_hbm.at[idx_v.at[g]], bufs[b], gsems[b]
            ).wait()
            pltpu.sync_copy(bufs[b], out_hbm.at[pl.ds(rbase + g * CHUNK, CHUNK)])

        for b in range(NBUF):
            fire(b, b)

        def body(o, carry):
            g0 = o * NBUF
            for b in range(NBUF):
                drain(g0 + b, b)
                fire(g0 + b + NBUF, b)
            return carry

        lax.fori_loop(0, G // NBUF - 1, body, 0)

        for b in range(NBUF):
            drain(G - NBUF + b, b)

    return k(table, idx2d)


def kernel(inputs, table):
    b, h = inputs.shape
    idx2d = inputs.astype(jnp.int32).reshape(-1, CHUNK)
    out = _sc_gather(idx2d, table)
    return out.reshape(b, h, D)
